# baseline (speedup 1.0000x reference)
import jax
import jax.numpy as jnp
from jax.experimental import pallas as pl
from jax.experimental.pallas import tpu as pltpu

_B, _S, _H = 4, 8192, 4096
_E = 64
_TOP_K = 8
_KERNEL = 4
_BS = 1024
_NJ = _S // _BS


def _router_body(x_ref, x2_ref, w_ref, b_ref, idx_ref, wt_ref, ytail, ybuf):
    j = pl.program_id(1)

    @pl.when(j == 0)
    def _():
        ytail[...] = jnp.zeros_like(ytail)

    @pl.when(j >= 1)
    def _():
        y = ybuf[...]
        prev = ytail[0:3, :]
        ycat = jnp.concatenate([prev, y], axis=0)
        logits = (ycat[3:3 + _BS, 3 * _E:4 * _E]
                  + ycat[2:2 + _BS, 2 * _E:3 * _E]
                  + ycat[1:1 + _BS, 1 * _E:2 * _E]
                  + ycat[0:_BS, 0:_E]) + b_ref[0]
        ytail[0:3, :] = y[_BS - 3:_BS, :]

        m = jnp.max(logits, axis=1, keepdims=True)
        e = jnp.exp(logits - m)
        z = jnp.sum(e, axis=1, keepdims=True)
        p = e / z

        iota = jax.lax.broadcasted_iota(jnp.int32, (_BS, _E), 1)
        bits = jax.lax.bitcast_convert_type(p, jnp.int32)
        keys = jax.lax.bitcast_convert_type(
            jnp.bitwise_or(jnp.bitwise_and(bits, -64), 63 - iota), jnp.float32)
        ks = []
        for _ in range(_TOP_K):
            kmax = jnp.max(keys, axis=1, keepdims=True)
            ks.append(kmax)
            keys = jnp.where(keys == kmax, -1.0, keys)
        kcat = jnp.concatenate(ks, axis=1)
        kbits = jax.lax.bitcast_convert_type(kcat, jnp.int32)
        idx_ref[0] = 63 - jnp.bitwise_and(kbits, 63)
        topv = jax.lax.bitcast_convert_type(
            jnp.bitwise_and(kbits, -64), jnp.float32)
        denom = jnp.sum(topv, axis=1, keepdims=True) + 1e-6
        wt_ref[0] = topv / denom

    @pl.when(j < _NJ)
    def _():
        w = w_ref[...]
        _HB = _BS // 2
        ybuf[0:_HB, :] = jnp.dot(x_ref[0], w,
                                 preferred_element_type=jnp.float32)
        ybuf[_HB:_BS, :] = jnp.dot(x2_ref[0], w,
                                   preferred_element_type=jnp.float32)


def kernel(hidden_states, gate_conv_w, bias):
    wpack = jnp.transpose(gate_conv_w, (1, 2, 0)).reshape(_H, _KERNEL * _E)
    bias2 = bias.reshape(1, _E).astype(jnp.float32)

    grid = (_B, _NJ + 1)
    out_shape = (
        jax.ShapeDtypeStruct((_B, _S, _TOP_K), jnp.int32),
        jax.ShapeDtypeStruct((_B, _S, _TOP_K), jnp.float32),
    )
    idx, wt = pl.pallas_call(
        _router_body,
        grid=grid,
        in_specs=[
            pl.BlockSpec((1, _BS // 2, _H),
                         lambda b, j: (b, 2 * jnp.minimum(j, _NJ - 1), 0)),
            pl.BlockSpec((1, _BS // 2, _H),
                         lambda b, j: (b, 2 * jnp.minimum(j, _NJ - 1) + 1, 0)),
            pl.BlockSpec((_H, _KERNEL * _E), lambda b, j: (0, 0)),
            pl.BlockSpec((1, _E), lambda b, j: (0, 0)),
        ],
        out_specs=(
            pl.BlockSpec((1, _BS, _TOP_K),
                         lambda b, j: (b, jnp.maximum(j - 1, 0), 0)),
            pl.BlockSpec((1, _BS, _TOP_K),
                         lambda b, j: (b, jnp.maximum(j - 1, 0), 0)),
        ),
        out_shape=out_shape,
        scratch_shapes=[
            pltpu.VMEM((8, _KERNEL * _E), jnp.float32),
            pltpu.VMEM((_BS, _KERNEL * _E), jnp.float32),
        ],
        compiler_params=pltpu.CompilerParams(
            dimension_semantics=("arbitrary", "arbitrary"),
        ),
    )(hidden_states, hidden_states, wpack, bias2)
    return idx, wt


# speedup vs baseline: 1.0746x; 1.0746x over previous
import jax
import jax.numpy as jnp
from jax.experimental import pallas as pl
from jax.experimental.pallas import tpu as pltpu

_B, _S, _H = 4, 8192, 4096
_E = 64
_TOP_K = 8
_KERNEL = 4
_BS = 1024
_NJ = _S // _BS
_NG = _B * _NJ


def _router_body(x_ref, w_ref, b_ref, idx_ref, wt_ref, ytail, ybuf):
    g = pl.program_id(0)

    @pl.when(g % _NJ == 1)
    def _():
        ytail[...] = jnp.zeros_like(ytail)

    @pl.when(g >= 1)
    def _():
        y = ybuf[...]
        prev = ytail[0:3, :]
        ycat = jnp.concatenate([prev, y], axis=0)
        logits = (ycat[3:3 + _BS, 3 * _E:4 * _E]
                  + ycat[2:2 + _BS, 2 * _E:3 * _E]
                  + ycat[1:1 + _BS, 1 * _E:2 * _E]
                  + ycat[0:_BS, 0:_E]) + b_ref[0]
        ytail[0:3, :] = y[_BS - 3:_BS, :]

        m = jnp.max(logits, axis=1, keepdims=True)
        e = jnp.exp(logits - m)
        z = jnp.sum(e, axis=1, keepdims=True)
        p = e / z

        iota = jax.lax.broadcasted_iota(jnp.int32, (_BS, _E), 1)
        bits = jax.lax.bitcast_convert_type(p, jnp.int32)
        keys = jax.lax.bitcast_convert_type(
            jnp.bitwise_or(jnp.bitwise_and(bits, -64), 63 - iota), jnp.float32)
        ks = []
        for _ in range(_TOP_K):
            kmax = jnp.max(keys, axis=1, keepdims=True)
            ks.append(kmax)
            keys = jnp.where(keys == kmax, -1.0, keys)
        kcat = jnp.concatenate(ks, axis=1)
        kbits = jax.lax.bitcast_convert_type(kcat, jnp.int32)
        idx_ref[0] = 63 - jnp.bitwise_and(kbits, 63)
        topv = jax.lax.bitcast_convert_type(
            jnp.bitwise_and(kbits, -64), jnp.float32)
        denom = jnp.sum(topv, axis=1, keepdims=True) + 1e-6
        wt_ref[0] = topv / denom

    @pl.when(g < _NG)
    def _():
        ybuf[...] = jnp.dot(x_ref[0], w_ref[...],
                            preferred_element_type=jnp.float32)


def kernel(hidden_states, gate_conv_w, bias):
    wpack = jnp.transpose(gate_conv_w, (1, 2, 0)).reshape(_H, _KERNEL * _E)
    bias2 = bias.reshape(1, _E).astype(jnp.float32)

    out_shape = (
        jax.ShapeDtypeStruct((_B, _S, _TOP_K), jnp.int32),
        jax.ShapeDtypeStruct((_B, _S, _TOP_K), jnp.float32),
    )

    def _xmap(g):
        gc = jnp.minimum(g, _NG - 1)
        return (gc // _NJ, gc % _NJ, 0)

    def _omap(g):
        gp = jnp.maximum(g - 1, 0)
        return (gp // _NJ, gp % _NJ, 0)

    idx, wt = pl.pallas_call(
        _router_body,
        grid=(_NG + 1,),
        in_specs=[
            pl.BlockSpec((1, _BS, _H), _xmap),
            pl.BlockSpec((_H, _KERNEL * _E), lambda g: (0, 0)),
            pl.BlockSpec((1, _E), lambda g: (0, 0)),
        ],
        out_specs=(
            pl.BlockSpec((1, _BS, _TOP_K), _omap),
            pl.BlockSpec((1, _BS, _TOP_K), _omap),
        ),
        out_shape=out_shape,
        scratch_shapes=[
            pltpu.VMEM((8, _KERNEL * _E), jnp.float32),
            pltpu.VMEM((_BS, _KERNEL * _E), jnp.float32),
        ],
        compiler_params=pltpu.CompilerParams(
            dimension_semantics=("arbitrary",),
        ),
    )(hidden_states, wpack, bias2)
    return idx, wt
